# baseline (speedup 1.0000x reference)
import jax
import jax.numpy as jnp
from jax.experimental import pallas as pl
from jax.experimental.pallas import tpu as pltpu

_BN_EPS = 1e-5


def _gap_bn_kernel(x_ref, scale_ref, shift_ref, out_ref):
    s = jnp.sum(x_ref[...], axis=-1, keepdims=True, dtype=jnp.float32)
    out_ref[...] = s * scale_ref[...] + shift_ref[...]


def kernel(x_nchw, gamma, beta, running_mean, running_var):
    n, c, h, w = x_nchw.shape
    hw = h * w
    x = x_nchw.reshape(n, c, hw)

    inv_std = jax.lax.rsqrt(running_var.astype(jnp.float32) + _BN_EPS)
    s0 = gamma.astype(jnp.float32) * inv_std
    scale = (s0 / hw).reshape(1, c, 1)
    shift = (beta.astype(jnp.float32)
             - running_mean.astype(jnp.float32) * s0).reshape(1, c, 1)

    nb = 4
    while n % nb:
        nb //= 2

    in_block = nb * c * hw * x.dtype.itemsize
    vmem_limit = int(min(48 << 20, 4 * in_block + (8 << 20)))

    out = pl.pallas_call(
        _gap_bn_kernel,
        out_shape=jax.ShapeDtypeStruct((n, c, 1), jnp.float32),
        grid=(n // nb,),
        in_specs=[
            pl.BlockSpec((nb, c, hw), lambda i: (i, 0, 0)),
            pl.BlockSpec((1, c, 1), lambda i: (0, 0, 0)),
            pl.BlockSpec((1, c, 1), lambda i: (0, 0, 0)),
        ],
        out_specs=pl.BlockSpec((nb, c, 1), lambda i: (i, 0, 0)),
        compiler_params=pltpu.CompilerParams(
            dimension_semantics=("parallel",),
            vmem_limit_bytes=vmem_limit,
        ),
        cost_estimate=pl.CostEstimate(
            flops=2 * n * c * hw,
            transcendentals=0,
            bytes_accessed=n * c * hw * x.dtype.itemsize + n * c * 4 + 2 * c * 4,
        ),
    )(x, scale, shift)
    return out.reshape(n, c)


# speedup vs baseline: 7.3075x; 7.3075x over previous
import jax
import jax.numpy as jnp
from jax.experimental import pallas as pl
from jax.experimental.pallas import tpu as pltpu

_BN_EPS = 1e-5


def _gap_bn_kernel(x_ref, scale_ref, shift_ref, out_ref):
    s = jnp.sum(x_ref[...], axis=1, keepdims=True, dtype=jnp.float32)
    out_ref[...] = s * scale_ref[...] + shift_ref[...]


def kernel(x_nchw, gamma, beta, running_mean, running_var):
    n, c, h, w = x_nchw.shape
    hw = h * w
    x = jnp.transpose(x_nchw, (0, 2, 3, 1)).reshape(n, hw, c)

    inv_std = jax.lax.rsqrt(running_var.astype(jnp.float32) + _BN_EPS)
    s0 = gamma.astype(jnp.float32) * inv_std
    scale = (s0 / hw).reshape(1, 1, c)
    shift = (beta.astype(jnp.float32)
             - running_mean.astype(jnp.float32) * s0).reshape(1, 1, c)

    nb = 8
    while n % nb:
        nb //= 2

    in_block = nb * hw * c * x.dtype.itemsize
    vmem_limit = int(min(48 << 20, 4 * in_block + (8 << 20)))

    out = pl.pallas_call(
        _gap_bn_kernel,
        out_shape=jax.ShapeDtypeStruct((n, 1, c), jnp.float32),
        grid=(n // nb,),
        in_specs=[
            pl.BlockSpec((nb, hw, c), lambda i: (i, 0, 0)),
            pl.BlockSpec((1, 1, c), lambda i: (0, 0, 0)),
            pl.BlockSpec((1, 1, c), lambda i: (0, 0, 0)),
        ],
        out_specs=pl.BlockSpec((nb, 1, c), lambda i: (i, 0, 0)),
        compiler_params=pltpu.CompilerParams(
            dimension_semantics=("parallel",),
            vmem_limit_bytes=vmem_limit,
        ),
        cost_estimate=pl.CostEstimate(
            flops=2 * n * c * hw,
            transcendentals=0,
            bytes_accessed=n * c * hw * x.dtype.itemsize + n * c * 4 + 2 * c * 4,
        ),
    )(x, scale, shift)
    return out.reshape(n, c)


# speedup vs baseline: 8.4816x; 1.1607x over previous
import jax
import jax.numpy as jnp
from jax.experimental import pallas as pl
from jax.experimental.pallas import tpu as pltpu

_BN_EPS = 1e-5


def _gap_bn_kernel(x_ref, scale_ref, shift_ref, out_ref):
    s = jnp.sum(x_ref[...], axis=1, keepdims=True, dtype=jnp.float32)
    out_ref[...] = s * scale_ref[...] + shift_ref[...]


def kernel(x_nchw, gamma, beta, running_mean, running_var):
    n, c, h, w = x_nchw.shape
    hw = h * w
    x = jnp.transpose(x_nchw, (0, 2, 3, 1)).reshape(n, hw, c)

    inv_std = jax.lax.rsqrt(running_var.astype(jnp.float32) + _BN_EPS)
    s0 = gamma.astype(jnp.float32) * inv_std
    scale = (s0 / hw).reshape(1, 1, c)
    shift = (beta.astype(jnp.float32)
             - running_mean.astype(jnp.float32) * s0).reshape(1, 1, c)

    nb = 16
    while n % nb:
        nb //= 2

    in_block = nb * hw * c * x.dtype.itemsize
    vmem_limit = int(min(48 << 20, 4 * in_block + (8 << 20)))

    out = pl.pallas_call(
        _gap_bn_kernel,
        out_shape=jax.ShapeDtypeStruct((n, 1, c), jnp.float32),
        grid=(n // nb,),
        in_specs=[
            pl.BlockSpec((nb, hw, c), lambda i: (i, 0, 0)),
            pl.BlockSpec((1, 1, c), lambda i: (0, 0, 0)),
            pl.BlockSpec((1, 1, c), lambda i: (0, 0, 0)),
        ],
        out_specs=pl.BlockSpec((nb, 1, c), lambda i: (i, 0, 0)),
        compiler_params=pltpu.CompilerParams(
            dimension_semantics=("parallel",),
            vmem_limit_bytes=vmem_limit,
        ),
        cost_estimate=pl.CostEstimate(
            flops=2 * n * c * hw,
            transcendentals=0,
            bytes_accessed=n * c * hw * x.dtype.itemsize + n * c * 4 + 2 * c * 4,
        ),
    )(x, scale, shift)
    return out.reshape(n, c)
